# baseline (speedup 1.0000x reference)
import dataclasses
import functools

import jax
import jax.numpy as jnp
from jax import lax
from jax.experimental import pallas as pl
from jax.experimental.pallas import tpu as pltpu
from jax.experimental.pallas import tpu_sc as plsc

B = 16384
K = 64
F_DIM = 128
N_CATS = 1000
NCAT_P = 1024
LAMBDA_W = 0.01
LAMBDA_E = 0.01

NC = 2
NS = 16
NW = NC * NS
B_PER_W = B // NW
CHUNK = 64
N_CHUNKS = B_PER_W // CHUNK
GRP = 16
N_GRP = CHUNK // GRP
RING = 10

BLK = 2048
NBLK = B // BLK


def _sc_gather(u, i, j, F, IC, PuT, Qi, Bi, Bc):
    mesh = plsc.VectorSubcoreMesh(core_axis_name="c", subcore_axis_name="s")
    f32 = jnp.float32
    i32 = jnp.int32
    out_type = (
        jax.ShapeDtypeStruct((B, K), f32),
        jax.ShapeDtypeStruct((B, K), f32),
        jax.ShapeDtypeStruct((B, K), f32),
        jax.ShapeDtypeStruct((B, F_DIM), f32),
        jax.ShapeDtypeStruct((B, F_DIM), f32),
        jax.ShapeDtypeStruct((B,), f32),
        jax.ShapeDtypeStruct((B,), f32),
        jax.ShapeDtypeStruct((B,), f32),
        jax.ShapeDtypeStruct((B,), f32),
        jax.ShapeDtypeStruct((B,), i32),
        jax.ShapeDtypeStruct((B,), i32),
    )

    cp = pltpu.CompilerParams()
    if "needs_layout_passes" in pltpu.CompilerParams.__dataclass_fields__:
        cp = dataclasses.replace(cp, needs_layout_passes=False)

    @functools.partial(
        pl.kernel,
        mesh=mesh,
        out_type=out_type,
        compiler_params=cp,
        scratch_types=[
            pltpu.VMEM((B_PER_W + GRP,), i32),
            pltpu.VMEM((B_PER_W,), i32),
            pltpu.VMEM((B_PER_W,), i32),
            pltpu.VMEM((CHUNK,), i32),
            pltpu.VMEM((CHUNK,), i32),
            pltpu.VMEM((RING, K, F_DIM), f32),
            pltpu.VMEM((CHUNK, K), f32),
            pltpu.VMEM((CHUNK, K), f32),
            pltpu.VMEM((CHUNK, K), f32),
            pltpu.VMEM((CHUNK, F_DIM), f32),
            pltpu.VMEM((CHUNK, F_DIM), f32),
            pltpu.VMEM((CHUNK,), f32),
            pltpu.VMEM((CHUNK,), f32),
            pltpu.VMEM((CHUNK,), f32),
            pltpu.VMEM((CHUNK,), f32),
            pltpu.SemaphoreType.DMA,
            pltpu.SemaphoreType.DMA,
            pltpu.SemaphoreType.DMA,
            pltpu.SemaphoreType.DMA,
            pltpu.SemaphoreType.DMA,
            pltpu.SemaphoreType.DMA,
            pltpu.SemaphoreType.DMA,
            pltpu.SemaphoreType.DMA,
            pltpu.SemaphoreType.DMA,
            pltpu.SemaphoreType.DMA,
            pltpu.SemaphoreType.DMA,
            pltpu.SemaphoreType.DMA,
            pltpu.SemaphoreType.DMA,
            pltpu.SemaphoreType.DMA,
        ],
    )
    def k(u_hbm, i_hbm, j_hbm, F_hbm, IC_hbm, PuT_hbm, Qi_hbm, Bi_hbm,
          Bc_hbm, pu_o, qi_o, qj_o, vi_o, vj_o, bi_o, bj_o, bic_o, bjc_o,
          ci_o, cj_o, u_v, i_v, j_v, ci_v, cj_v, blk_v, pu_v, qi_v, qj_v,
          vi_v, vj_v, bi_v, bj_v, bic_v, bjc_v, sem_g, sem_c, sem_d, sem_w,
          *ring_sems):
        wid = lax.axis_index("s") * NC + lax.axis_index("c")
        base = wid * B_PER_W
        pltpu.sync_copy(u_hbm.at[pl.ds(base, B_PER_W)],
                        u_v.at[pl.ds(0, B_PER_W)])
        pltpu.sync_copy(i_hbm.at[pl.ds(base, B_PER_W)], i_v)
        pltpu.sync_copy(j_hbm.at[pl.ds(base, B_PER_W)], j_v)

        kio = lax.broadcasted_iota(i32, (GRP,), 0)

        def fire_pu(idx_scalar, slot):
            c128 = pl.multiple_of((idx_scalar // F_DIM) * F_DIM, F_DIM)
            return pltpu.async_copy(
                PuT_hbm.at[:, pl.ds(c128, F_DIM)], blk_v.at[slot],
                ring_sems[slot])

        for c in range(N_CHUNKS):
            off = base + c * CHUNK
            iw = i_v.at[pl.ds(c * CHUNK, CHUNK)]
            jw = j_v.at[pl.ds(c * CHUNK, CHUNK)]

            h_ci = pltpu.async_copy(IC_hbm.at[iw], ci_v, sem_c)
            h_cj = pltpu.async_copy(IC_hbm.at[jw], cj_v, sem_c)
            hs = [
                pltpu.async_copy(Bi_hbm.at[iw], bi_v, sem_g),
                pltpu.async_copy(Bi_hbm.at[jw], bj_v, sem_g),
                pltpu.async_copy(F_hbm.at[iw], vi_v, sem_g),
                pltpu.async_copy(F_hbm.at[jw], vj_v, sem_g),
            ]

            @pl.loop(0, N_GRP)
            def _(g):
                src = c * CHUNK + g * GRP
                idxi = i_v[pl.ds(src, GRP)]
                idxj = j_v[pl.ds(src, GRP)]
                for l in range(GRP):
                    dst = g * GRP + l
                    pltpu.async_copy(
                        Qi_hbm.at[idxi[l]], qi_v.at[dst], sem_d)
                    pltpu.async_copy(
                        Qi_hbm.at[idxj[l]], qj_v.at[dst], sem_d)

            uvec0 = u_v[pl.ds(c * CHUNK, GRP)]
            for l in range(RING):
                fire_pu(uvec0[l], l)

            @pl.loop(0, N_GRP)
            def _(g):
                r0 = g * GRP
                uvec = u_v[pl.ds(c * CHUNK + r0, GRP)]
                unext = u_v[pl.ds(c * CHUNK + r0 + GRP, GRP)]
                for l in range(GRP):
                    slot = l % RING
                    s_next = r0 + l + RING
                    pltpu.make_async_copy(
                        PuT_hbm.at[:, pl.ds(0, F_DIM)], blk_v.at[slot],
                        ring_sems[slot]).wait()
                    idx = uvec[l]
                    lane = jnp.full((GRP,), idx % F_DIM, i32)
                    for q in range(K // GRP):
                        vals = plsc.load_gather(
                            blk_v.at[slot], [kio + q * GRP, lane])
                        pu_v[r0 + l, pl.ds(q * GRP, GRP)] = vals
                    if l + RING < GRP:
                        nidx = uvec[l + RING]
                    else:
                        nidx = unext[l + RING - GRP]

                    @pl.when(s_next < CHUNK)
                    def _():
                        fire_pu(nidx, slot)

            h_ci.wait()
            h_cj.wait()
            hs += [
                pltpu.async_copy(Bc_hbm.at[ci_v], bic_v, sem_g),
                pltpu.async_copy(Bc_hbm.at[cj_v], bjc_v, sem_g),
            ]
            for h in hs:
                h.wait()
            @pl.loop(0, CHUNK)
            def _(r):
                pltpu.make_async_copy(
                    Qi_hbm.at[0], qi_v.at[0], sem_d).wait()
                pltpu.make_async_copy(
                    Qi_hbm.at[0], qj_v.at[0], sem_d).wait()

            ws = [
                pltpu.async_copy(pu_v, pu_o.at[pl.ds(off, CHUNK)], sem_w),
                pltpu.async_copy(qi_v, qi_o.at[pl.ds(off, CHUNK)], sem_w),
                pltpu.async_copy(qj_v, qj_o.at[pl.ds(off, CHUNK)], sem_w),
                pltpu.async_copy(vi_v, vi_o.at[pl.ds(off, CHUNK)], sem_w),
                pltpu.async_copy(vj_v, vj_o.at[pl.ds(off, CHUNK)], sem_w),
                pltpu.async_copy(bi_v, bi_o.at[pl.ds(off, CHUNK)], sem_w),
                pltpu.async_copy(bj_v, bj_o.at[pl.ds(off, CHUNK)], sem_w),
                pltpu.async_copy(bic_v, bic_o.at[pl.ds(off, CHUNK)], sem_w),
                pltpu.async_copy(bjc_v, bjc_o.at[pl.ds(off, CHUNK)], sem_w),
                pltpu.async_copy(ci_v, ci_o.at[pl.ds(off, CHUNK)], sem_w),
                pltpu.async_copy(cj_v, cj_o.at[pl.ds(off, CHUNK)], sem_w),
            ]
            for h in ws:
                h.wait()

    return k(u, i, j, F, IC, PuT, Qi, Bi, Bc)


def _tc_body(pu, qi, qj, vi, vj, bi, bj, bic, bjc, ci, cj, e_ref, bp_ref,
             ic_ref, loss_o, auc_o):
    b = pl.program_id(0)
    dv = (vi[...] - vj[...]) * (1.0 / 60.0)
    dq = qi[...] - qj[...]

    ci_row = ci[0, 0, :].reshape(1, BLK)
    cj_row = cj[0, 0, :].reshape(1, BLK)
    cats = lax.broadcasted_iota(jnp.int32, (NCAT_P, BLK), 0)
    ohT_i = (ci_row == cats).astype(jnp.float32)
    ohT_j = (cj_row == cats).astype(jnp.float32)
    cdims = (((0,), (0,)), ((), ()))
    ii = lax.dot_general(ohT_i, ic_ref[...], cdims,
                         preferred_element_type=jnp.float32)
    ij = lax.dot_general(ohT_j, ic_ref[...], cdims,
                         preferred_element_type=jnp.float32)

    t = jnp.dot(dv, e_ref[...], preferred_element_type=jnp.float32) \
        + dq - (ii - ij)
    s = jnp.sum(pu[...] * t, axis=1)
    dvbp = jnp.sum(dv * bp_ref[...], axis=1)
    bterm = bi[0, 0, :] - bj[0, 0, :] + bic[0, 0, :] - bjc[0, 0, :]
    y = bterm + s + dvbp

    ll = jnp.sum(jnp.log1p(jnp.exp(-y)))
    auc_p = jnp.sum((y > 0).astype(jnp.float32))

    def ssq(x):
        return jnp.sum(x[...] * x[...])

    reg_w = 0.5 * (ssq(pu) + ssq(qi) + ssq(qj) + ssq(ii) + ssq(ij))
    reg_b = 0.5 * (ssq(bi) + ssq(bj) + ssq(bic) + ssq(bjc))
    partial = ll + LAMBDA_W * (reg_w + reg_b)

    @pl.when(b == 0)
    def _():
        loss_o[0, 0] = LAMBDA_E * 0.5 * (ssq(e_ref) + ssq(bp_ref))
        auc_o[0, 0] = 0.0

    loss_o[0, 0] += partial
    auc_o[0, 0] += auc_p


def _tc_compute(pu, qi, qj, vi, vj, bi3, bj3, bic3, bjc3, ci3, cj3, E,
                bp_row, ic_pad):
    f32 = jnp.float32
    k_spec = pl.BlockSpec((BLK, K), lambda b: (b, 0))
    f_spec = pl.BlockSpec((BLK, F_DIM), lambda b: (b, 0))
    s_spec = pl.BlockSpec((1, 1, BLK), lambda b: (b, 0, 0))
    e_spec = pl.BlockSpec((F_DIM, K), lambda b: (0, 0))
    bp_spec = pl.BlockSpec((1, F_DIM), lambda b: (0, 0))
    ic_spec = pl.BlockSpec((NCAT_P, K), lambda b: (0, 0))
    out_spec = pl.BlockSpec(memory_space=pltpu.SMEM)
    return pl.pallas_call(
        _tc_body,
        grid=(NBLK,),
        in_specs=[k_spec, k_spec, k_spec, f_spec, f_spec,
                  s_spec, s_spec, s_spec, s_spec, s_spec, s_spec,
                  e_spec, bp_spec, ic_spec],
        out_specs=[out_spec, out_spec],
        out_shape=[jax.ShapeDtypeStruct((1, 1), f32),
                   jax.ShapeDtypeStruct((1, 1), f32)],
    )(pu, qi, qj, vi, vj, bi3, bj3, bic3, bjc3, ci3, cj3, E, bp_row, ic_pad)


def kernel(u, i, j, F, IC, Pu, Qi, Bi, E, Bp, Ic, Bc):
    u = u.astype(jnp.int32)
    i = i.astype(jnp.int32)
    j = j.astype(jnp.int32)
    (pu, qi, qj, vi, vj, bi, bj, bic, bjc, ci, cj) = _sc_gather(
        u, i, j, F, IC, Pu.T, Qi, Bi, Bc)
    bi3 = bi.reshape(NBLK, 1, BLK)
    bj3 = bj.reshape(NBLK, 1, BLK)
    bic3 = bic.reshape(NBLK, 1, BLK)
    bjc3 = bjc.reshape(NBLK, 1, BLK)
    ci3 = ci.reshape(NBLK, 1, BLK)
    cj3 = cj.reshape(NBLK, 1, BLK)
    bp_row = Bp.reshape(1, F_DIM)
    ic_pad = jnp.pad(Ic, ((0, NCAT_P - N_CATS), (0, 0)))
    loss, auc = _tc_compute(pu, qi, qj, vi, vj, bi3, bj3, bic3, bjc3,
                            ci3, cj3, E, bp_row, ic_pad)
    return (loss[0, 0], auc[0, 0])


# speedup vs baseline: 1.0346x; 1.0346x over previous
import dataclasses
import functools

import jax
import jax.numpy as jnp
from jax import lax
from jax.experimental import pallas as pl
from jax.experimental.pallas import tpu as pltpu
from jax.experimental.pallas import tpu_sc as plsc

B = 16384
K = 64
F_DIM = 128
N_CATS = 1000
NCAT_P = 1024
LAMBDA_W = 0.01
LAMBDA_E = 0.01

NC = 2
NS = 16
NW = NC * NS
B_PER_W = B // NW
CHUNK = 64
N_CHUNKS = B_PER_W // CHUNK
GRP = 16
N_GRP = CHUNK // GRP
RING = 8

BLK = 2048
NBLK = B // BLK

_f32 = jnp.float32
_i32 = jnp.int32


def _sc_cp():
    cp = pltpu.CompilerParams()
    if "needs_layout_passes" in pltpu.CompilerParams.__dataclass_fields__:
        cp = dataclasses.replace(cp, needs_layout_passes=False)
    return cp


def _sc_gather(i, j, F, IC, Qi, Bi, Bc):
    mesh = plsc.VectorSubcoreMesh(core_axis_name="c", subcore_axis_name="s")
    out_type = (
        jax.ShapeDtypeStruct((B, K), _f32),
        jax.ShapeDtypeStruct((B, K), _f32),
        jax.ShapeDtypeStruct((B, F_DIM), _f32),
        jax.ShapeDtypeStruct((B, F_DIM), _f32),
        jax.ShapeDtypeStruct((B,), _f32),
        jax.ShapeDtypeStruct((B,), _f32),
        jax.ShapeDtypeStruct((B,), _f32),
        jax.ShapeDtypeStruct((B,), _f32),
        jax.ShapeDtypeStruct((B,), _i32),
        jax.ShapeDtypeStruct((B,), _i32),
    )

    @functools.partial(
        pl.kernel,
        mesh=mesh,
        out_type=out_type,
        compiler_params=_sc_cp(),
        scratch_types=[
            pltpu.VMEM((B_PER_W,), _i32),
            pltpu.VMEM((B_PER_W,), _i32),
            pltpu.VMEM((CHUNK,), _i32),
            pltpu.VMEM((CHUNK,), _i32),
            pltpu.VMEM((CHUNK, K), _f32),
            pltpu.VMEM((CHUNK, K), _f32),
            pltpu.VMEM((CHUNK, F_DIM), _f32),
            pltpu.VMEM((CHUNK, F_DIM), _f32),
            pltpu.VMEM((CHUNK,), _f32),
            pltpu.VMEM((CHUNK,), _f32),
            pltpu.VMEM((CHUNK,), _f32),
            pltpu.VMEM((CHUNK,), _f32),
            pltpu.SemaphoreType.DMA,
            pltpu.SemaphoreType.DMA,
            pltpu.SemaphoreType.DMA,
            pltpu.SemaphoreType.DMA,
        ],
    )
    def k(i_hbm, j_hbm, F_hbm, IC_hbm, Qi_hbm, Bi_hbm, Bc_hbm,
          qi_o, qj_o, vi_o, vj_o, bi_o, bj_o, bic_o, bjc_o, ci_o, cj_o,
          i_v, j_v, ci_v, cj_v, qi_v, qj_v, vi_v, vj_v, bi_v, bj_v,
          bic_v, bjc_v, sem_g, sem_c, sem_d, sem_w):
        wid = lax.axis_index("s") * NC + lax.axis_index("c")
        base = wid * B_PER_W
        pltpu.sync_copy(i_hbm.at[pl.ds(base, B_PER_W)], i_v)
        pltpu.sync_copy(j_hbm.at[pl.ds(base, B_PER_W)], j_v)

        for c in range(N_CHUNKS):
            off = base + c * CHUNK
            iw = i_v.at[pl.ds(c * CHUNK, CHUNK)]
            jw = j_v.at[pl.ds(c * CHUNK, CHUNK)]

            h_ci = pltpu.async_copy(IC_hbm.at[iw], ci_v, sem_c)
            h_cj = pltpu.async_copy(IC_hbm.at[jw], cj_v, sem_c)
            hs = [
                pltpu.async_copy(Bi_hbm.at[iw], bi_v, sem_g),
                pltpu.async_copy(Bi_hbm.at[jw], bj_v, sem_g),
                pltpu.async_copy(F_hbm.at[iw], vi_v, sem_g),
                pltpu.async_copy(F_hbm.at[jw], vj_v, sem_g),
            ]

            @pl.loop(0, N_GRP)
            def _(g):
                src = c * CHUNK + g * GRP
                idxi = i_v[pl.ds(src, GRP)]
                idxj = j_v[pl.ds(src, GRP)]
                for l in range(GRP):
                    dst = g * GRP + l
                    pltpu.async_copy(
                        Qi_hbm.at[idxi[l]], qi_v.at[dst], sem_d)
                    pltpu.async_copy(
                        Qi_hbm.at[idxj[l]], qj_v.at[dst], sem_d)

            h_ci.wait()
            h_cj.wait()
            hs += [
                pltpu.async_copy(Bc_hbm.at[ci_v], bic_v, sem_g),
                pltpu.async_copy(Bc_hbm.at[cj_v], bjc_v, sem_g),
            ]
            for h in hs:
                h.wait()
            @pl.loop(0, CHUNK)
            def _(r):
                pltpu.make_async_copy(
                    Qi_hbm.at[0], qi_v.at[0], sem_d).wait()
                pltpu.make_async_copy(
                    Qi_hbm.at[0], qj_v.at[0], sem_d).wait()

            ws = [
                pltpu.async_copy(qi_v, qi_o.at[pl.ds(off, CHUNK)], sem_w),
                pltpu.async_copy(qj_v, qj_o.at[pl.ds(off, CHUNK)], sem_w),
                pltpu.async_copy(vi_v, vi_o.at[pl.ds(off, CHUNK)], sem_w),
                pltpu.async_copy(vj_v, vj_o.at[pl.ds(off, CHUNK)], sem_w),
                pltpu.async_copy(bi_v, bi_o.at[pl.ds(off, CHUNK)], sem_w),
                pltpu.async_copy(bj_v, bj_o.at[pl.ds(off, CHUNK)], sem_w),
                pltpu.async_copy(bic_v, bic_o.at[pl.ds(off, CHUNK)], sem_w),
                pltpu.async_copy(bjc_v, bjc_o.at[pl.ds(off, CHUNK)], sem_w),
                pltpu.async_copy(ci_v, ci_o.at[pl.ds(off, CHUNK)], sem_w),
                pltpu.async_copy(cj_v, cj_o.at[pl.ds(off, CHUNK)], sem_w),
            ]
            for h in ws:
                h.wait()

    return k(i, j, F, IC, Qi, Bi, Bc)


def _sc_gather_pu(u, PuT):
    mesh = plsc.VectorSubcoreMesh(core_axis_name="c", subcore_axis_name="s")

    @functools.partial(
        pl.kernel,
        mesh=mesh,
        out_type=jax.ShapeDtypeStruct((B, K), _f32),
        compiler_params=_sc_cp(),
        scratch_types=[
            pltpu.VMEM((B_PER_W + GRP,), _i32),
            pltpu.VMEM((RING, K, F_DIM), _f32),
            pltpu.VMEM((CHUNK, K), _f32),
            pltpu.SemaphoreType.DMA,
        ] + [pltpu.SemaphoreType.DMA] * RING,
    )
    def k(u_hbm, PuT_hbm, pu_o, u_v, blk_v, pu_v, sem_w, *ring_sems):
        wid = lax.axis_index("s") * NC + lax.axis_index("c")
        base = wid * B_PER_W
        pltpu.sync_copy(u_hbm.at[pl.ds(base, B_PER_W)],
                        u_v.at[pl.ds(0, B_PER_W)])

        kio = lax.broadcasted_iota(_i32, (GRP,), 0)

        def fire_pu(idx_scalar, slot):
            c128 = pl.multiple_of((idx_scalar // F_DIM) * F_DIM, F_DIM)
            return pltpu.async_copy(
                PuT_hbm.at[:, pl.ds(c128, F_DIM)], blk_v.at[slot],
                ring_sems[slot])

        for c in range(N_CHUNKS):
            off = base + c * CHUNK
            uvec0 = u_v[pl.ds(c * CHUNK, GRP)]
            for l in range(RING):
                fire_pu(uvec0[l], l)

            @pl.loop(0, N_GRP)
            def _(g):
                r0 = g * GRP
                uvec = u_v[pl.ds(c * CHUNK + r0, GRP)]
                unext = u_v[pl.ds(c * CHUNK + r0 + GRP, GRP)]
                for l in range(GRP):
                    slot = l % RING
                    s_next = r0 + l + RING
                    pltpu.make_async_copy(
                        PuT_hbm.at[:, pl.ds(0, F_DIM)], blk_v.at[slot],
                        ring_sems[slot]).wait()
                    idx = uvec[l]
                    lane = jnp.full((GRP,), idx % F_DIM, _i32)
                    for q in range(K // GRP):
                        vals = plsc.load_gather(
                            blk_v.at[slot], [kio + q * GRP, lane])
                        pu_v[r0 + l, pl.ds(q * GRP, GRP)] = vals
                    if l + RING < GRP:
                        nidx = uvec[l + RING]
                    else:
                        nidx = unext[l + RING - GRP]

                    @pl.when(s_next < CHUNK)
                    def _():
                        fire_pu(nidx, slot)

            h = pltpu.async_copy(pu_v, pu_o.at[pl.ds(off, CHUNK)], sem_w)
            h.wait()

    return k(u, PuT)


def _tc_a_body(qi, qj, vi, vj, bi, bj, bic, bjc, ci, cj, e_ref, bp_ref,
               ic_ref, t_o, y0_o, rega_o):
    b = pl.program_id(0)
    dv = (vi[...] - vj[...]) * (1.0 / 60.0)
    dq = qi[...] - qj[...]

    ci_row = ci[0, 0, :].reshape(1, BLK)
    cj_row = cj[0, 0, :].reshape(1, BLK)
    cats = lax.broadcasted_iota(_i32, (NCAT_P, BLK), 0)
    ohT_i = (ci_row == cats).astype(_f32)
    ohT_j = (cj_row == cats).astype(_f32)
    cdims = (((0,), (0,)), ((), ()))
    ii = lax.dot_general(ohT_i, ic_ref[...], cdims,
                         preferred_element_type=_f32)
    ij = lax.dot_general(ohT_j, ic_ref[...], cdims,
                         preferred_element_type=_f32)

    t = jnp.dot(dv, e_ref[...], preferred_element_type=_f32) \
        + dq - (ii - ij)
    t_o[...] = t
    dvbp = jnp.sum(dv * bp_ref[...], axis=1)
    bterm = bi[0, 0, :] - bj[0, 0, :] + bic[0, 0, :] - bjc[0, 0, :]
    y0_o[0, 0, :] = bterm + dvbp

    def ssq(x):
        return jnp.sum(x[...] * x[...])

    reg_w = 0.5 * (ssq(qi) + ssq(qj) + ssq(ii) + ssq(ij))
    reg_b = 0.5 * (ssq(bi) + ssq(bj) + ssq(bic) + ssq(bjc))
    partial = LAMBDA_W * (reg_w + reg_b)

    @pl.when(b == 0)
    def _():
        rega_o[0, 0] = LAMBDA_E * 0.5 * (ssq(e_ref) + ssq(bp_ref))

    rega_o[0, 0] += partial


def _tc_a(qi, qj, vi, vj, bi3, bj3, bic3, bjc3, ci3, cj3, E, bp_row, ic_pad):
    k_spec = pl.BlockSpec((BLK, K), lambda b: (b, 0))
    f_spec = pl.BlockSpec((BLK, F_DIM), lambda b: (b, 0))
    s_spec = pl.BlockSpec((1, 1, BLK), lambda b: (b, 0, 0))
    e_spec = pl.BlockSpec((F_DIM, K), lambda b: (0, 0))
    bp_spec = pl.BlockSpec((1, F_DIM), lambda b: (0, 0))
    ic_spec = pl.BlockSpec((NCAT_P, K), lambda b: (0, 0))
    smem_spec = pl.BlockSpec(memory_space=pltpu.SMEM)
    return pl.pallas_call(
        _tc_a_body,
        grid=(NBLK,),
        in_specs=[k_spec, k_spec, f_spec, f_spec,
                  s_spec, s_spec, s_spec, s_spec, s_spec, s_spec,
                  e_spec, bp_spec, ic_spec],
        out_specs=[k_spec, s_spec, smem_spec],
        out_shape=[jax.ShapeDtypeStruct((B, K), _f32),
                   jax.ShapeDtypeStruct((NBLK, 1, BLK), _f32),
                   jax.ShapeDtypeStruct((1, 1), _f32)],
    )(qi, qj, vi, vj, bi3, bj3, bic3, bjc3, ci3, cj3, E, bp_row, ic_pad)


def _tc_b_body(pu, t, y0, rega, loss_o, auc_o):
    b = pl.program_id(0)
    s = jnp.sum(pu[...] * t[...], axis=1)
    y = y0[0, 0, :] + s
    ll = jnp.sum(jnp.log1p(jnp.exp(-y)))
    auc_p = jnp.sum((y > 0).astype(_f32))
    reg_pu = LAMBDA_W * 0.5 * jnp.sum(pu[...] * pu[...])

    @pl.when(b == 0)
    def _():
        loss_o[0, 0] = rega[0, 0]
        auc_o[0, 0] = 0.0

    loss_o[0, 0] += ll + reg_pu
    auc_o[0, 0] += auc_p


def _tc_b(pu, t, y03, rega):
    k_spec = pl.BlockSpec((BLK, K), lambda b: (b, 0))
    s_spec = pl.BlockSpec((1, 1, BLK), lambda b: (b, 0, 0))
    smem_spec = pl.BlockSpec(memory_space=pltpu.SMEM)
    return pl.pallas_call(
        _tc_b_body,
        grid=(NBLK,),
        in_specs=[k_spec, k_spec, s_spec, smem_spec],
        out_specs=[smem_spec, smem_spec],
        out_shape=[jax.ShapeDtypeStruct((1, 1), _f32),
                   jax.ShapeDtypeStruct((1, 1), _f32)],
    )(pu, t, y03, rega)


def kernel(u, i, j, F, IC, Pu, Qi, Bi, E, Bp, Ic, Bc):
    u = u.astype(_i32)
    i = i.astype(_i32)
    j = j.astype(_i32)
    (qi, qj, vi, vj, bi, bj, bic, bjc, ci, cj) = _sc_gather(
        i, j, F, IC, Qi, Bi, Bc)
    pu = _sc_gather_pu(u, Pu.T)
    bi3 = bi.reshape(NBLK, 1, BLK)
    bj3 = bj.reshape(NBLK, 1, BLK)
    bic3 = bic.reshape(NBLK, 1, BLK)
    bjc3 = bjc.reshape(NBLK, 1, BLK)
    ci3 = ci.reshape(NBLK, 1, BLK)
    cj3 = cj.reshape(NBLK, 1, BLK)
    bp_row = Bp.reshape(1, F_DIM)
    ic_pad = jnp.pad(Ic, ((0, NCAT_P - N_CATS), (0, 0)))
    t, y03, rega = _tc_a(qi, qj, vi, vj, bi3, bj3, bic3, bjc3, ci3, cj3,
                         E, bp_row, ic_pad)
    loss, auc = _tc_b(pu, t, y03, rega)
    return (loss[0, 0], auc[0, 0])


# speedup vs baseline: 1.0710x; 1.0352x over previous
import dataclasses
import functools

import jax
import jax.numpy as jnp
from jax import lax
from jax.experimental import pallas as pl
from jax.experimental.pallas import tpu as pltpu
from jax.experimental.pallas import tpu_sc as plsc

B = 16384
K = 64
F_DIM = 128
N_CATS = 1000
NCAT_P = 1024
LAMBDA_W = 0.01
LAMBDA_E = 0.01

NC = 2
NS = 16
NW = NC * NS
B_PER_W = B // NW
CHUNK = 64
N_CHUNKS = B_PER_W // CHUNK
GRP = 16
N_GRP = CHUNK // GRP
RING = 8

BLK = 2048
NBLK = B // BLK

_f32 = jnp.float32
_i32 = jnp.int32


def _sc_cp():
    cp = pltpu.CompilerParams()
    if "needs_layout_passes" in pltpu.CompilerParams.__dataclass_fields__:
        cp = dataclasses.replace(cp, needs_layout_passes=False)
    return cp


def _sc_gather(i, j, F, IC, Bi, Bc):
    mesh = plsc.VectorSubcoreMesh(core_axis_name="c", subcore_axis_name="s")
    out_type = (
        jax.ShapeDtypeStruct((B, F_DIM), _f32),
        jax.ShapeDtypeStruct((B, F_DIM), _f32),
        jax.ShapeDtypeStruct((B,), _f32),
        jax.ShapeDtypeStruct((B,), _f32),
        jax.ShapeDtypeStruct((B,), _f32),
        jax.ShapeDtypeStruct((B,), _f32),
        jax.ShapeDtypeStruct((B,), _i32),
        jax.ShapeDtypeStruct((B,), _i32),
    )

    @functools.partial(
        pl.kernel,
        mesh=mesh,
        out_type=out_type,
        compiler_params=_sc_cp(),
        scratch_types=[
            pltpu.VMEM((B_PER_W,), _i32),
            pltpu.VMEM((B_PER_W,), _i32),
            pltpu.VMEM((CHUNK,), _i32),
            pltpu.VMEM((CHUNK,), _i32),
            pltpu.VMEM((CHUNK, F_DIM), _f32),
            pltpu.VMEM((CHUNK, F_DIM), _f32),
            pltpu.VMEM((CHUNK,), _f32),
            pltpu.VMEM((CHUNK,), _f32),
            pltpu.VMEM((CHUNK,), _f32),
            pltpu.VMEM((CHUNK,), _f32),
            pltpu.SemaphoreType.DMA,
            pltpu.SemaphoreType.DMA,
            pltpu.SemaphoreType.DMA,
        ],
    )
    def k(i_hbm, j_hbm, F_hbm, IC_hbm, Bi_hbm, Bc_hbm,
          vi_o, vj_o, bi_o, bj_o, bic_o, bjc_o, ci_o, cj_o,
          i_v, j_v, ci_v, cj_v, vi_v, vj_v, bi_v, bj_v,
          bic_v, bjc_v, sem_g, sem_c, sem_w):
        wid = lax.axis_index("s") * NC + lax.axis_index("c")
        base = wid * B_PER_W
        pltpu.sync_copy(i_hbm.at[pl.ds(base, B_PER_W)], i_v)
        pltpu.sync_copy(j_hbm.at[pl.ds(base, B_PER_W)], j_v)

        for c in range(N_CHUNKS):
            off = base + c * CHUNK
            iw = i_v.at[pl.ds(c * CHUNK, CHUNK)]
            jw = j_v.at[pl.ds(c * CHUNK, CHUNK)]

            h_ci = pltpu.async_copy(IC_hbm.at[iw], ci_v, sem_c)
            h_cj = pltpu.async_copy(IC_hbm.at[jw], cj_v, sem_c)
            hs = [
                pltpu.async_copy(Bi_hbm.at[iw], bi_v, sem_g),
                pltpu.async_copy(Bi_hbm.at[jw], bj_v, sem_g),
                pltpu.async_copy(F_hbm.at[iw], vi_v, sem_g),
                pltpu.async_copy(F_hbm.at[jw], vj_v, sem_g),
            ]

            h_ci.wait()
            h_cj.wait()
            hs += [
                pltpu.async_copy(Bc_hbm.at[ci_v], bic_v, sem_g),
                pltpu.async_copy(Bc_hbm.at[cj_v], bjc_v, sem_g),
            ]
            for h in hs:
                h.wait()

            ws = [
                pltpu.async_copy(vi_v, vi_o.at[pl.ds(off, CHUNK)], sem_w),
                pltpu.async_copy(vj_v, vj_o.at[pl.ds(off, CHUNK)], sem_w),
                pltpu.async_copy(bi_v, bi_o.at[pl.ds(off, CHUNK)], sem_w),
                pltpu.async_copy(bj_v, bj_o.at[pl.ds(off, CHUNK)], sem_w),
                pltpu.async_copy(bic_v, bic_o.at[pl.ds(off, CHUNK)], sem_w),
                pltpu.async_copy(bjc_v, bjc_o.at[pl.ds(off, CHUNK)], sem_w),
                pltpu.async_copy(ci_v, ci_o.at[pl.ds(off, CHUNK)], sem_w),
                pltpu.async_copy(cj_v, cj_o.at[pl.ds(off, CHUNK)], sem_w),
            ]
            for h in ws:
                h.wait()

    return k(i, j, F, IC, Bi, Bc)


def _sc_gather_pu(u, i, j, PuT, Qi):
    mesh = plsc.VectorSubcoreMesh(core_axis_name="c", subcore_axis_name="s")
    out_type = (
        jax.ShapeDtypeStruct((B, K), _f32),
        jax.ShapeDtypeStruct((B, K), _f32),
        jax.ShapeDtypeStruct((B, K), _f32),
    )

    @functools.partial(
        pl.kernel,
        mesh=mesh,
        out_type=out_type,
        compiler_params=_sc_cp(),
        scratch_types=[
            pltpu.VMEM((B_PER_W + GRP,), _i32),
            pltpu.VMEM((B_PER_W,), _i32),
            pltpu.VMEM((B_PER_W,), _i32),
            pltpu.VMEM((RING, K, F_DIM), _f32),
            pltpu.VMEM((CHUNK, K), _f32),
            pltpu.VMEM((CHUNK, K), _f32),
            pltpu.VMEM((CHUNK, K), _f32),
            pltpu.SemaphoreType.DMA,
            pltpu.SemaphoreType.DMA,
        ] + [pltpu.SemaphoreType.DMA] * RING,
    )
    def k(u_hbm, i_hbm, j_hbm, PuT_hbm, Qi_hbm, pu_o, qi_o, qj_o,
          u_v, i_v, j_v, blk_v, pu_v, qi_v, qj_v, sem_d, sem_w, *ring_sems):
        wid = lax.axis_index("s") * NC + lax.axis_index("c")
        base = wid * B_PER_W
        pltpu.sync_copy(u_hbm.at[pl.ds(base, B_PER_W)],
                        u_v.at[pl.ds(0, B_PER_W)])
        pltpu.sync_copy(i_hbm.at[pl.ds(base, B_PER_W)], i_v)
        pltpu.sync_copy(j_hbm.at[pl.ds(base, B_PER_W)], j_v)

        kio = lax.broadcasted_iota(_i32, (GRP,), 0)

        def fire_pu(idx_scalar, slot):
            c128 = pl.multiple_of((idx_scalar // F_DIM) * F_DIM, F_DIM)
            return pltpu.async_copy(
                PuT_hbm.at[:, pl.ds(c128, F_DIM)], blk_v.at[slot],
                ring_sems[slot])

        for c in range(N_CHUNKS):
            off = base + c * CHUNK

            @pl.loop(0, N_GRP)
            def _(g):
                src = c * CHUNK + g * GRP
                idxi = i_v[pl.ds(src, GRP)]
                idxj = j_v[pl.ds(src, GRP)]
                for l in range(GRP):
                    dst = g * GRP + l
                    pltpu.async_copy(
                        Qi_hbm.at[idxi[l]], qi_v.at[dst], sem_d)
                    pltpu.async_copy(
                        Qi_hbm.at[idxj[l]], qj_v.at[dst], sem_d)

            uvec0 = u_v[pl.ds(c * CHUNK, GRP)]
            for l in range(RING):
                fire_pu(uvec0[l], l)

            @pl.loop(0, N_GRP)
            def _(g):
                r0 = g * GRP
                uvec = u_v[pl.ds(c * CHUNK + r0, GRP)]
                unext = u_v[pl.ds(c * CHUNK + r0 + GRP, GRP)]
                for l in range(GRP):
                    slot = l % RING
                    s_next = r0 + l + RING
                    pltpu.make_async_copy(
                        PuT_hbm.at[:, pl.ds(0, F_DIM)], blk_v.at[slot],
                        ring_sems[slot]).wait()
                    idx = uvec[l]
                    lane = jnp.full((GRP,), idx % F_DIM, _i32)
                    for q in range(K // GRP):
                        vals = plsc.load_gather(
                            blk_v.at[slot], [kio + q * GRP, lane])
                        pu_v[r0 + l, pl.ds(q * GRP, GRP)] = vals
                    if l + RING < GRP:
                        nidx = uvec[l + RING]
                    else:
                        nidx = unext[l + RING - GRP]

                    @pl.when(s_next < CHUNK)
                    def _():
                        fire_pu(nidx, slot)

            @pl.loop(0, CHUNK)
            def _(r):
                pltpu.make_async_copy(
                    Qi_hbm.at[0], qi_v.at[0], sem_d).wait()
                pltpu.make_async_copy(
                    Qi_hbm.at[0], qj_v.at[0], sem_d).wait()

            ws = [
                pltpu.async_copy(pu_v, pu_o.at[pl.ds(off, CHUNK)], sem_w),
                pltpu.async_copy(qi_v, qi_o.at[pl.ds(off, CHUNK)], sem_w),
                pltpu.async_copy(qj_v, qj_o.at[pl.ds(off, CHUNK)], sem_w),
            ]
            for h in ws:
                h.wait()

    return k(u, i, j, PuT, Qi)


def _tc_a_body(vi, vj, bi, bj, bic, bjc, ci, cj, e_ref, bp_ref,
               ic_ref, t_o, y0_o, rega_o):
    b = pl.program_id(0)
    dv = (vi[...] - vj[...]) * (1.0 / 60.0)

    ci_row = ci[0, 0, :].reshape(1, BLK)
    cj_row = cj[0, 0, :].reshape(1, BLK)
    cats = lax.broadcasted_iota(_i32, (NCAT_P, BLK), 0)
    ohT_i = (ci_row == cats).astype(_f32)
    ohT_j = (cj_row == cats).astype(_f32)
    cdims = (((0,), (0,)), ((), ()))
    ii = lax.dot_general(ohT_i, ic_ref[...], cdims,
                         preferred_element_type=_f32)
    ij = lax.dot_general(ohT_j, ic_ref[...], cdims,
                         preferred_element_type=_f32)

    t = jnp.dot(dv, e_ref[...], preferred_element_type=_f32) \
        - (ii - ij)
    t_o[...] = t
    dvbp = jnp.sum(dv * bp_ref[...], axis=1)
    bterm = bi[0, 0, :] - bj[0, 0, :] + bic[0, 0, :] - bjc[0, 0, :]
    y0_o[0, 0, :] = bterm + dvbp

    def ssq(x):
        return jnp.sum(x[...] * x[...])

    reg_w = 0.5 * (ssq(ii) + ssq(ij))
    reg_b = 0.5 * (ssq(bi) + ssq(bj) + ssq(bic) + ssq(bjc))
    partial = LAMBDA_W * (reg_w + reg_b)

    @pl.when(b == 0)
    def _():
        rega_o[0, 0] = LAMBDA_E * 0.5 * (ssq(e_ref) + ssq(bp_ref))

    rega_o[0, 0] += partial


def _tc_a(vi, vj, bi3, bj3, bic3, bjc3, ci3, cj3, E, bp_row, ic_pad):
    k_spec = pl.BlockSpec((BLK, K), lambda b: (b, 0))
    f_spec = pl.BlockSpec((BLK, F_DIM), lambda b: (b, 0))
    s_spec = pl.BlockSpec((1, 1, BLK), lambda b: (b, 0, 0))
    e_spec = pl.BlockSpec((F_DIM, K), lambda b: (0, 0))
    bp_spec = pl.BlockSpec((1, F_DIM), lambda b: (0, 0))
    ic_spec = pl.BlockSpec((NCAT_P, K), lambda b: (0, 0))
    smem_spec = pl.BlockSpec(memory_space=pltpu.SMEM)
    return pl.pallas_call(
        _tc_a_body,
        grid=(NBLK,),
        in_specs=[f_spec, f_spec,
                  s_spec, s_spec, s_spec, s_spec, s_spec, s_spec,
                  e_spec, bp_spec, ic_spec],
        out_specs=[k_spec, s_spec, smem_spec],
        out_shape=[jax.ShapeDtypeStruct((B, K), _f32),
                   jax.ShapeDtypeStruct((NBLK, 1, BLK), _f32),
                   jax.ShapeDtypeStruct((1, 1), _f32)],
    )(vi, vj, bi3, bj3, bic3, bjc3, ci3, cj3, E, bp_row, ic_pad)


def _tc_b_body(pu, qi, qj, t, y0, rega, loss_o, auc_o):
    b = pl.program_id(0)
    s = jnp.sum(pu[...] * (t[...] + qi[...] - qj[...]), axis=1)
    y = y0[0, 0, :] + s
    ll = jnp.sum(jnp.log1p(jnp.exp(-y)))
    auc_p = jnp.sum((y > 0).astype(_f32))

    def ssq(x):
        return jnp.sum(x[...] * x[...])

    reg_pu = LAMBDA_W * 0.5 * (ssq(pu) + ssq(qi) + ssq(qj))

    @pl.when(b == 0)
    def _():
        loss_o[0, 0] = rega[0, 0]
        auc_o[0, 0] = 0.0

    loss_o[0, 0] += ll + reg_pu
    auc_o[0, 0] += auc_p


def _tc_b(pu, qi, qj, t, y03, rega):
    k_spec = pl.BlockSpec((BLK, K), lambda b: (b, 0))
    s_spec = pl.BlockSpec((1, 1, BLK), lambda b: (b, 0, 0))
    smem_spec = pl.BlockSpec(memory_space=pltpu.SMEM)
    return pl.pallas_call(
        _tc_b_body,
        grid=(NBLK,),
        in_specs=[k_spec, k_spec, k_spec, k_spec, s_spec, smem_spec],
        out_specs=[smem_spec, smem_spec],
        out_shape=[jax.ShapeDtypeStruct((1, 1), _f32),
                   jax.ShapeDtypeStruct((1, 1), _f32)],
    )(pu, qi, qj, t, y03, rega)


def kernel(u, i, j, F, IC, Pu, Qi, Bi, E, Bp, Ic, Bc):
    u = u.astype(_i32)
    i = i.astype(_i32)
    j = j.astype(_i32)
    (vi, vj, bi, bj, bic, bjc, ci, cj) = _sc_gather(i, j, F, IC, Bi, Bc)
    pu, qi, qj = _sc_gather_pu(u, i, j, Pu.T, Qi)
    bi3 = bi.reshape(NBLK, 1, BLK)
    bj3 = bj.reshape(NBLK, 1, BLK)
    bic3 = bic.reshape(NBLK, 1, BLK)
    bjc3 = bjc.reshape(NBLK, 1, BLK)
    ci3 = ci.reshape(NBLK, 1, BLK)
    cj3 = cj.reshape(NBLK, 1, BLK)
    bp_row = Bp.reshape(1, F_DIM)
    ic_pad = jnp.pad(Ic, ((0, NCAT_P - N_CATS), (0, 0)))
    t, y03, rega = _tc_a(vi, vj, bi3, bj3, bic3, bjc3, ci3, cj3,
                         E, bp_row, ic_pad)
    loss, auc = _tc_b(pu, qi, qj, t, y03, rega)
    return (loss[0, 0], auc[0, 0])


# speedup vs baseline: 1.0972x; 1.0245x over previous
import dataclasses
import functools

import jax
import jax.numpy as jnp
from jax import lax
from jax.experimental import pallas as pl
from jax.experimental.pallas import tpu as pltpu
from jax.experimental.pallas import tpu_sc as plsc

B = 16384
K = 64
F_DIM = 128
N_CATS = 1000
NCAT_P = 1024
LAMBDA_W = 0.01
LAMBDA_E = 0.01

NC = 2
NS = 16
NW = NC * NS
B_PER_W = B // NW
CHUNK = 128
N_CHUNKS = B_PER_W // CHUNK
GRP = 16
N_GRP = CHUNK // GRP
RING = 8

BLK = 2048
NBLK = B // BLK

_f32 = jnp.float32
_i32 = jnp.int32


def _sc_cp():
    cp = pltpu.CompilerParams()
    if "needs_layout_passes" in pltpu.CompilerParams.__dataclass_fields__:
        cp = dataclasses.replace(cp, needs_layout_passes=False)
    return cp


def _sc_gather(i, j, F, IC, Bi, Bc):
    mesh = plsc.VectorSubcoreMesh(core_axis_name="c", subcore_axis_name="s")
    out_type = (
        jax.ShapeDtypeStruct((B, F_DIM), _f32),
        jax.ShapeDtypeStruct((B, F_DIM), _f32),
        jax.ShapeDtypeStruct((B,), _f32),
        jax.ShapeDtypeStruct((B,), _f32),
        jax.ShapeDtypeStruct((B,), _f32),
        jax.ShapeDtypeStruct((B,), _f32),
        jax.ShapeDtypeStruct((B,), _i32),
        jax.ShapeDtypeStruct((B,), _i32),
    )

    @functools.partial(
        pl.kernel,
        mesh=mesh,
        out_type=out_type,
        compiler_params=_sc_cp(),
        scratch_types=[
            pltpu.VMEM((B_PER_W,), _i32),
            pltpu.VMEM((B_PER_W,), _i32),
            pltpu.VMEM((CHUNK,), _i32),
            pltpu.VMEM((CHUNK,), _i32),
            pltpu.VMEM((CHUNK, F_DIM), _f32),
            pltpu.VMEM((CHUNK, F_DIM), _f32),
            pltpu.VMEM((CHUNK,), _f32),
            pltpu.VMEM((CHUNK,), _f32),
            pltpu.VMEM((CHUNK,), _f32),
            pltpu.VMEM((CHUNK,), _f32),
            pltpu.SemaphoreType.DMA,
            pltpu.SemaphoreType.DMA,
            pltpu.SemaphoreType.DMA,
        ],
    )
    def k(i_hbm, j_hbm, F_hbm, IC_hbm, Bi_hbm, Bc_hbm,
          vi_o, vj_o, bi_o, bj_o, bic_o, bjc_o, ci_o, cj_o,
          i_v, j_v, ci_v, cj_v, vi_v, vj_v, bi_v, bj_v,
          bic_v, bjc_v, sem_g, sem_c, sem_w):
        wid = lax.axis_index("s") * NC + lax.axis_index("c")
        base = wid * B_PER_W
        pltpu.sync_copy(i_hbm.at[pl.ds(base, B_PER_W)], i_v)
        pltpu.sync_copy(j_hbm.at[pl.ds(base, B_PER_W)], j_v)

        for c in range(N_CHUNKS):
            off = base + c * CHUNK
            iw = i_v.at[pl.ds(c * CHUNK, CHUNK)]
            jw = j_v.at[pl.ds(c * CHUNK, CHUNK)]

            h_ci = pltpu.async_copy(IC_hbm.at[iw], ci_v, sem_c)
            h_cj = pltpu.async_copy(IC_hbm.at[jw], cj_v, sem_c)
            hs = [
                pltpu.async_copy(Bi_hbm.at[iw], bi_v, sem_g),
                pltpu.async_copy(Bi_hbm.at[jw], bj_v, sem_g),
                pltpu.async_copy(F_hbm.at[iw], vi_v, sem_g),
                pltpu.async_copy(F_hbm.at[jw], vj_v, sem_g),
            ]

            h_ci.wait()
            h_cj.wait()
            hs += [
                pltpu.async_copy(Bc_hbm.at[ci_v], bic_v, sem_g),
                pltpu.async_copy(Bc_hbm.at[cj_v], bjc_v, sem_g),
            ]
            for h in hs:
                h.wait()

            ws = [
                pltpu.async_copy(vi_v, vi_o.at[pl.ds(off, CHUNK)], sem_w),
                pltpu.async_copy(vj_v, vj_o.at[pl.ds(off, CHUNK)], sem_w),
                pltpu.async_copy(bi_v, bi_o.at[pl.ds(off, CHUNK)], sem_w),
                pltpu.async_copy(bj_v, bj_o.at[pl.ds(off, CHUNK)], sem_w),
                pltpu.async_copy(bic_v, bic_o.at[pl.ds(off, CHUNK)], sem_w),
                pltpu.async_copy(bjc_v, bjc_o.at[pl.ds(off, CHUNK)], sem_w),
                pltpu.async_copy(ci_v, ci_o.at[pl.ds(off, CHUNK)], sem_w),
                pltpu.async_copy(cj_v, cj_o.at[pl.ds(off, CHUNK)], sem_w),
            ]
            for h in ws:
                h.wait()

    return k(i, j, F, IC, Bi, Bc)


def _sc_gather_pu(u, i, j, PuT, Qi):
    mesh = plsc.VectorSubcoreMesh(core_axis_name="c", subcore_axis_name="s")
    out_type = (
        jax.ShapeDtypeStruct((B, K), _f32),
        jax.ShapeDtypeStruct((B, K), _f32),
        jax.ShapeDtypeStruct((B, K), _f32),
    )

    @functools.partial(
        pl.kernel,
        mesh=mesh,
        out_type=out_type,
        compiler_params=_sc_cp(),
        scratch_types=[
            pltpu.VMEM((B_PER_W + GRP,), _i32),
            pltpu.VMEM((B_PER_W,), _i32),
            pltpu.VMEM((B_PER_W,), _i32),
            pltpu.VMEM((RING, K, F_DIM), _f32),
            pltpu.VMEM((CHUNK, K), _f32),
            pltpu.VMEM((CHUNK, K), _f32),
            pltpu.VMEM((CHUNK, K), _f32),
            pltpu.SemaphoreType.DMA,
            pltpu.SemaphoreType.DMA,
        ] + [pltpu.SemaphoreType.DMA] * RING,
    )
    def k(u_hbm, i_hbm, j_hbm, PuT_hbm, Qi_hbm, pu_o, qi_o, qj_o,
          u_v, i_v, j_v, blk_v, pu_v, qi_v, qj_v, sem_d, sem_w, *ring_sems):
        wid = lax.axis_index("s") * NC + lax.axis_index("c")
        base = wid * B_PER_W
        pltpu.sync_copy(u_hbm.at[pl.ds(base, B_PER_W)],
                        u_v.at[pl.ds(0, B_PER_W)])
        pltpu.sync_copy(i_hbm.at[pl.ds(base, B_PER_W)], i_v)
        pltpu.sync_copy(j_hbm.at[pl.ds(base, B_PER_W)], j_v)

        kio = lax.broadcasted_iota(_i32, (GRP,), 0)

        def fire_pu(idx_scalar, slot):
            c128 = pl.multiple_of((idx_scalar // F_DIM) * F_DIM, F_DIM)
            return pltpu.async_copy(
                PuT_hbm.at[:, pl.ds(c128, F_DIM)], blk_v.at[slot],
                ring_sems[slot])

        for c in range(N_CHUNKS):
            off = base + c * CHUNK

            @pl.loop(0, N_GRP)
            def _(g):
                src = c * CHUNK + g * GRP
                idxi = i_v[pl.ds(src, GRP)]
                idxj = j_v[pl.ds(src, GRP)]
                for l in range(GRP):
                    dst = g * GRP + l
                    pltpu.async_copy(
                        Qi_hbm.at[idxi[l]], qi_v.at[dst], sem_d)
                    pltpu.async_copy(
                        Qi_hbm.at[idxj[l]], qj_v.at[dst], sem_d)

            uvec0 = u_v[pl.ds(c * CHUNK, GRP)]
            for l in range(RING):
                fire_pu(uvec0[l], l)

            @pl.loop(0, N_GRP)
            def _(g):
                r0 = g * GRP
                uvec = u_v[pl.ds(c * CHUNK + r0, GRP)]
                unext = u_v[pl.ds(c * CHUNK + r0 + GRP, GRP)]
                for l in range(GRP):
                    slot = l % RING
                    s_next = r0 + l + RING
                    pltpu.make_async_copy(
                        PuT_hbm.at[:, pl.ds(0, F_DIM)], blk_v.at[slot],
                        ring_sems[slot]).wait()
                    idx = uvec[l]
                    lane = jnp.full((GRP,), idx % F_DIM, _i32)
                    for q in range(K // GRP):
                        vals = plsc.load_gather(
                            blk_v.at[slot], [kio + q * GRP, lane])
                        pu_v[r0 + l, pl.ds(q * GRP, GRP)] = vals
                    if l + RING < GRP:
                        nidx = uvec[l + RING]
                    else:
                        nidx = unext[l + RING - GRP]

                    @pl.when(s_next < CHUNK)
                    def _():
                        fire_pu(nidx, slot)

            @pl.loop(0, CHUNK)
            def _(r):
                pltpu.make_async_copy(
                    Qi_hbm.at[0], qi_v.at[0], sem_d).wait()
                pltpu.make_async_copy(
                    Qi_hbm.at[0], qj_v.at[0], sem_d).wait()

            ws = [
                pltpu.async_copy(pu_v, pu_o.at[pl.ds(off, CHUNK)], sem_w),
                pltpu.async_copy(qi_v, qi_o.at[pl.ds(off, CHUNK)], sem_w),
                pltpu.async_copy(qj_v, qj_o.at[pl.ds(off, CHUNK)], sem_w),
            ]
            for h in ws:
                h.wait()

    return k(u, i, j, PuT, Qi)


def _tc_a_body(vi, vj, bi, bj, bic, bjc, ci, cj, e_ref, bp_ref,
               ic_ref, t_o, y0_o, rega_o):
    b = pl.program_id(0)
    dv = (vi[...] - vj[...]) * (1.0 / 60.0)

    ci_row = ci[0, 0, :].reshape(1, BLK)
    cj_row = cj[0, 0, :].reshape(1, BLK)
    cats = lax.broadcasted_iota(_i32, (NCAT_P, BLK), 0)
    ohT_i = (ci_row == cats).astype(_f32)
    ohT_j = (cj_row == cats).astype(_f32)
    cdims = (((0,), (0,)), ((), ()))
    ii = lax.dot_general(ohT_i, ic_ref[...], cdims,
                         preferred_element_type=_f32)
    ij = lax.dot_general(ohT_j, ic_ref[...], cdims,
                         preferred_element_type=_f32)

    t = jnp.dot(dv, e_ref[...], preferred_element_type=_f32) \
        - (ii - ij)
    t_o[...] = t
    dvbp = jnp.sum(dv * bp_ref[...], axis=1)
    bterm = bi[0, 0, :] - bj[0, 0, :] + bic[0, 0, :] - bjc[0, 0, :]
    y0_o[0, 0, :] = bterm + dvbp

    def ssq(x):
        return jnp.sum(x[...] * x[...])

    reg_w = 0.5 * (ssq(ii) + ssq(ij))
    reg_b = 0.5 * (ssq(bi) + ssq(bj) + ssq(bic) + ssq(bjc))
    partial = LAMBDA_W * (reg_w + reg_b)

    @pl.when(b == 0)
    def _():
        rega_o[0, 0] = LAMBDA_E * 0.5 * (ssq(e_ref) + ssq(bp_ref))

    rega_o[0, 0] += partial


def _tc_a(vi, vj, bi3, bj3, bic3, bjc3, ci3, cj3, E, bp_row, ic_pad):
    k_spec = pl.BlockSpec((BLK, K), lambda b: (b, 0))
    f_spec = pl.BlockSpec((BLK, F_DIM), lambda b: (b, 0))
    s_spec = pl.BlockSpec((1, 1, BLK), lambda b: (b, 0, 0))
    e_spec = pl.BlockSpec((F_DIM, K), lambda b: (0, 0))
    bp_spec = pl.BlockSpec((1, F_DIM), lambda b: (0, 0))
    ic_spec = pl.BlockSpec((NCAT_P, K), lambda b: (0, 0))
    smem_spec = pl.BlockSpec(memory_space=pltpu.SMEM)
    return pl.pallas_call(
        _tc_a_body,
        grid=(NBLK,),
        in_specs=[f_spec, f_spec,
                  s_spec, s_spec, s_spec, s_spec, s_spec, s_spec,
                  e_spec, bp_spec, ic_spec],
        out_specs=[k_spec, s_spec, smem_spec],
        out_shape=[jax.ShapeDtypeStruct((B, K), _f32),
                   jax.ShapeDtypeStruct((NBLK, 1, BLK), _f32),
                   jax.ShapeDtypeStruct((1, 1), _f32)],
    )(vi, vj, bi3, bj3, bic3, bjc3, ci3, cj3, E, bp_row, ic_pad)


def _tc_b_body(pu, qi, qj, t, y0, rega, loss_o, auc_o):
    b = pl.program_id(0)
    s = jnp.sum(pu[...] * (t[...] + qi[...] - qj[...]), axis=1)
    y = y0[0, 0, :] + s
    ll = jnp.sum(jnp.log1p(jnp.exp(-y)))
    auc_p = jnp.sum((y > 0).astype(_f32))

    def ssq(x):
        return jnp.sum(x[...] * x[...])

    reg_pu = LAMBDA_W * 0.5 * (ssq(pu) + ssq(qi) + ssq(qj))

    @pl.when(b == 0)
    def _():
        loss_o[0, 0] = rega[0, 0]
        auc_o[0, 0] = 0.0

    loss_o[0, 0] += ll + reg_pu
    auc_o[0, 0] += auc_p


def _tc_b(pu, qi, qj, t, y03, rega):
    k_spec = pl.BlockSpec((BLK, K), lambda b: (b, 0))
    s_spec = pl.BlockSpec((1, 1, BLK), lambda b: (b, 0, 0))
    smem_spec = pl.BlockSpec(memory_space=pltpu.SMEM)
    return pl.pallas_call(
        _tc_b_body,
        grid=(NBLK,),
        in_specs=[k_spec, k_spec, k_spec, k_spec, s_spec, smem_spec],
        out_specs=[smem_spec, smem_spec],
        out_shape=[jax.ShapeDtypeStruct((1, 1), _f32),
                   jax.ShapeDtypeStruct((1, 1), _f32)],
    )(pu, qi, qj, t, y03, rega)


def kernel(u, i, j, F, IC, Pu, Qi, Bi, E, Bp, Ic, Bc):
    u = u.astype(_i32)
    i = i.astype(_i32)
    j = j.astype(_i32)
    (vi, vj, bi, bj, bic, bjc, ci, cj) = _sc_gather(i, j, F, IC, Bi, Bc)
    pu, qi, qj = _sc_gather_pu(u, i, j, Pu.T, Qi)
    bi3 = bi.reshape(NBLK, 1, BLK)
    bj3 = bj.reshape(NBLK, 1, BLK)
    bic3 = bic.reshape(NBLK, 1, BLK)
    bjc3 = bjc.reshape(NBLK, 1, BLK)
    ci3 = ci.reshape(NBLK, 1, BLK)
    cj3 = cj.reshape(NBLK, 1, BLK)
    bp_row = Bp.reshape(1, F_DIM)
    ic_pad = jnp.pad(Ic, ((0, NCAT_P - N_CATS), (0, 0)))
    t, y03, rega = _tc_a(vi, vj, bi3, bj3, bic3, bjc3, ci3, cj3,
                         E, bp_row, ic_pad)
    loss, auc = _tc_b(pu, qi, qj, t, y03, rega)
    return (loss[0, 0], auc[0, 0])
